# Initial kernel scaffold; baseline (speedup 1.0000x reference)
import jax
import jax.numpy as jnp
from jax.experimental import pallas as pl


def kernel(x, W, b, codebook):
    raise NotImplementedError("write your pallas kernel here")


# speedup vs baseline: 1.0826x; 1.0826x over previous
import functools

import jax
import jax.numpy as jnp
from jax import lax
from jax.experimental import pallas as pl
from jax.experimental.pallas import tpu as pltpu
from jax.experimental.pallas import tpu_sc as plsc

_VOCAB = 1024
_COMM = 64
_NUM_COMMS = 8
_BETA = 0.25

_ROWS_BLK = 256

_NC = 2
_NS = 16
_NW = _NC * _NS
_GATHER_W = 128


def _vq_tc_body(x_ref, w_ref, b_ref, cb_ref, idx_ref, dsum_ref):
    first = (pl.program_id(0) == 0) & (pl.program_id(1) == 0)
    x = x_ref[...]
    cbt = jnp.transpose(cb_ref[...], (1, 0))
    cnorm = jnp.sum(cbt * cbt, axis=0, keepdims=True)
    wg = w_ref[0]
    bg = b_ref[0, 0, :]
    fg = lax.dot_general(x, wg, (((1,), (1,)), ((), ())),
                         preferred_element_type=jnp.float32)
    fg = fg + bg[None, :]
    cross = lax.dot_general(fg, cbt, (((1,), (0,)), ((), ())),
                            preferred_element_type=jnp.float32)
    scores = cnorm - 2.0 * cross
    minval = jnp.min(scores, axis=1, keepdims=True)
    iota = lax.broadcasted_iota(jnp.int32, scores.shape, 1)
    idx = jnp.min(jnp.where(scores == minval, iota, _VOCAB), axis=1)
    idx_ref[0, 0, 0, :] = idx
    fnorm = jnp.sum(fg * fg, axis=1)
    partial = jnp.sum(minval[:, 0] + fnorm)

    @pl.when(first)
    def _init():
        dsum_ref[...] = jnp.zeros_like(dsum_ref)

    dsum_ref[...] += partial.reshape(1, 1)


def _vq_argmin(xf, W3, b3, codebook):
    rows = xf.shape[0]
    grid = rows // _ROWS_BLK
    return pl.pallas_call(
        _vq_tc_body,
        grid=(grid, _NUM_COMMS),
        in_specs=[
            pl.BlockSpec((_ROWS_BLK, xf.shape[1]), lambda i, j: (i, 0)),
            pl.BlockSpec((1, _COMM, W3.shape[2]), lambda i, j: (j, 0, 0)),
            pl.BlockSpec((1, 1, _COMM), lambda i, j: (j, 0, 0)),
            pl.BlockSpec(codebook.shape, lambda i, j: (0, 0)),
        ],
        out_specs=[
            pl.BlockSpec((1, 1, 1, _ROWS_BLK), lambda i, j: (i, j, 0, 0)),
            pl.BlockSpec((1, 1), lambda i, j: (0, 0)),
        ],
        out_shape=[
            jax.ShapeDtypeStruct((grid, _NUM_COMMS, 1, _ROWS_BLK), jnp.int32),
            jax.ShapeDtypeStruct((1, 1), jnp.float32),
        ],
    )(xf, W3, b3, codebook)


def _sc_gather_body(cb_hbm, idx_hbm, out_hbm, idx_v, rows_v, sem):
    wid = lax.axis_index("s") * _NC + lax.axis_index("c")
    n_rows = idx_hbm.shape[0]
    per_w = n_rows // _NW
    rbase = wid * per_w
    pltpu.sync_copy(idx_hbm.at[pl.ds(rbase, per_w)], idx_v)
    for g in range(per_w):
        pltpu.async_copy(cb_hbm.at[idx_v.at[g]], rows_v, sem).wait()
        pltpu.sync_copy(rows_v, out_hbm.at[pl.ds((rbase + g) * _GATHER_W,
                                                 _GATHER_W)])


def _sc_gather(codebook, idx2):
    tokens = idx2.shape[0] * idx2.shape[1]
    per_w = idx2.shape[0] // _NW
    fn = functools.partial(
        pl.kernel,
        out_type=jax.ShapeDtypeStruct((tokens, _COMM), jnp.float32),
        mesh=plsc.VectorSubcoreMesh(core_axis_name="c", subcore_axis_name="s"),
        compiler_params=pltpu.CompilerParams(use_tc_tiling_on_sc=False),
        scratch_types=[
            pltpu.VMEM((per_w, _GATHER_W), jnp.int32),
            pltpu.VMEM((_GATHER_W, _COMM), jnp.float32),
            pltpu.SemaphoreType.DMA,
        ],
    )(_sc_gather_body)
    return fn(codebook, idx2)


def kernel(x, W, b, codebook):
    B, T, N, H = x.shape
    xf = x.reshape(B * T * N, H)
    W3 = W.reshape(_NUM_COMMS, _COMM, H)
    b3 = b.reshape(_NUM_COMMS, 1, _COMM)
    idx4, dsum = _vq_argmin(xf, W3, b3, codebook)
    tokens = B * T * N * _NUM_COMMS
    idx2 = jnp.transpose(idx4[:, :, 0, :], (0, 2, 1)).reshape(
        tokens // _GATHER_W, _GATHER_W)
    qflat = _sc_gather(codebook, idx2)
    vq_loss = (1.0 + _BETA) * dsum[0, 0] / (tokens * _COMM)
    comm_output = qflat.reshape(B, N, _NUM_COMMS, _COMM)
    comm_log_probs = jnp.zeros((B * T, N), dtype=jnp.float32)
    return (comm_output, comm_output, comm_log_probs, vq_loss)
